# baseline (speedup 1.0000x reference)
import functools

import jax
import jax.numpy as jnp
from jax import lax
from jax.experimental import pallas as pl
from jax.experimental.pallas import tpu as pltpu
from jax.experimental.pallas import tpu_sc as plsc

B = 16384
D = 64
DP = 2 * D

_info = plsc.get_sparse_core_info()
_NC, _NS = _info.num_cores, _info.num_subcores
_NW = _NC * _NS
_BPW = B // _NW
_CHUNK = 128
_NCHUNK = _BPW // _CHUNK
_HALFC = _NCHUNK // 2
_IROWS = 16

_mesh = plsc.VectorSubcoreMesh(core_axis_name="c", subcore_axis_name="s")


def _pair_transpose(tT, C):
    cols = tT.shape[1]
    nblk = (cols + C - 1) // C

    S = 4096 if (C // 2) % 4096 == 0 else C // 2

    def body(t_ref, out_ref):
        for b in range(C // 2 // S):
            lo, hi = b * S, (b + 1) * S
            trl = jnp.transpose(t_ref[:, lo:hi])
            trr = jnp.transpose(t_ref[:, C // 2 + lo:C // 2 + hi])
            out_ref[lo:hi, :] = jnp.concatenate([trl, trr], axis=1)

    return pl.pallas_call(
        body,
        grid=(nblk,),
        in_specs=[pl.BlockSpec((D, C), lambda i: (0, i))],
        out_specs=pl.BlockSpec((C // 2, 128), lambda i: (i, 0)),
        out_shape=jax.ShapeDtypeStruct((nblk * C // 2, 128), jnp.float32),
    )(tT)


def _pair_index(r, C):
    half = C // 2
    p = (r // C) * half + (r % half)
    side = (r // half) % 2
    return p, side


@functools.partial(
    pl.kernel,
    mesh=_mesh,
    compiler_params=pltpu.CompilerParams(use_tc_tiling_on_sc=True),
    out_type=[jax.ShapeDtypeStruct((B, DP), jnp.float32)] * 3,
    scratch_types=[
        pltpu.VMEM((_IROWS, _CHUNK), jnp.int32),
        pltpu.VMEM((_HALFC * _CHUNK, DP), jnp.float32),
        pltpu.VMEM((_HALFC * _CHUNK, DP), jnp.float32),
        pltpu.VMEM((_HALFC * _CHUNK, DP), jnp.float32),
        pltpu.SemaphoreType.DMA,
        pltpu.SemaphoreType.DMA,
        pltpu.SemaphoreType.DMA,
    ],
)
def _sc_gather(idx_all, t0, t1, t2, o0, o1, o2,
               iv, r0, r1, r2, s0, s1, s2):
    wid = lax.axis_index("s") * _NC + lax.axis_index("c")
    base = wid * _BPW
    pltpu.sync_copy(idx_all.at[wid], iv)
    for h in range(2):
        copies = []
        for j in range(_HALFC):
            c = h * _HALFC + j
            dst = pl.ds(j * _CHUNK, _CHUNK)
            copies.append(pltpu.async_copy(t0.at[iv.at[c]], r0.at[dst], s0))
            copies.append(
                pltpu.async_copy(t1.at[iv.at[_NCHUNK + c]], r1.at[dst], s1))
            copies.append(
                pltpu.async_copy(t2.at[iv.at[2 * _NCHUNK + c]], r2.at[dst], s2))
        for c in copies:
            c.wait()
        span = pl.ds(base + h * _HALFC * _CHUNK, _HALFC * _CHUNK)
        pltpu.sync_copy(r0, o0.at[span])
        pltpu.sync_copy(r1, o1.at[span])
        pltpu.sync_copy(r2, o2.at[span])


_BM = 1024


def _mlp_body(sb, e0, e1, e2, w1, b1, w2, b2, out):
    side = sb[...]
    sel = []
    for k, e in enumerate((e0, e1, e2)):
        p = (side[:, k] == 1).reshape(-1, 1)
        sel.append(jnp.where(p, e[:, D:], e[:, :D]))
    emb = jnp.concatenate(sel, axis=1)
    h = jnp.dot(emb, w1[...], preferred_element_type=jnp.float32) + b1[...]
    h = jnp.maximum(h, 0.0)
    out[...] = jnp.dot(h, w2[...], preferred_element_type=jnp.float32) + b2[...]


def _tc_mlp(sb, e0, e1, e2, W1, b1, W2, b2):
    n1, n2 = W1.shape[1], W2.shape[1]
    grid = (B // _BM,)
    return pl.pallas_call(
        _mlp_body,
        grid=grid,
        in_specs=[
            pl.BlockSpec((_BM, 3), lambda i: (i, 0)),
            pl.BlockSpec((_BM, DP), lambda i: (i, 0)),
            pl.BlockSpec((_BM, DP), lambda i: (i, 0)),
            pl.BlockSpec((_BM, DP), lambda i: (i, 0)),
            pl.BlockSpec((3 * D, n1), lambda i: (0, 0)),
            pl.BlockSpec((1, n1), lambda i: (0, 0)),
            pl.BlockSpec((n1, n2), lambda i: (0, 0)),
            pl.BlockSpec((1, n2), lambda i: (0, 0)),
        ],
        out_specs=pl.BlockSpec((_BM, n2), lambda i: (i, 0)),
        out_shape=jax.ShapeDtypeStruct((B, n2), jnp.float32),
    )(sb, e0, e1, e2, W1, b1.reshape(1, n1), W2, b2.reshape(1, n2))


_C_ITEM = 32768
_C_CAT = 1000
_C_BRAND = 10000


def kernel(x, table_item_id, table_category, table_brand, W1, b1, W2, b2):
    xi = x.astype(jnp.int32)
    p0, s0 = _pair_index(xi[:, 0], _C_ITEM)
    p1, s1 = _pair_index(xi[:, 1], _C_CAT)
    p2, s2 = _pair_index(xi[:, 2], _C_BRAND)
    idx0 = p0.reshape(_NW, _NCHUNK, _CHUNK)
    idx1 = p1.reshape(_NW, _NCHUNK, _CHUNK)
    idx2 = p2.reshape(_NW, _NCHUNK, _CHUNK)
    pad = jnp.zeros((_NW, _IROWS - 3 * _NCHUNK, _CHUNK), jnp.int32)
    idx_all = jnp.concatenate([idx0, idx1, idx2, pad], axis=1)
    sb = jnp.stack([s0, s1, s2], axis=1)
    t0 = _pair_transpose(table_item_id.T, _C_ITEM)
    t1 = _pair_transpose(table_category.T, _C_CAT)
    t2 = _pair_transpose(table_brand.T, _C_BRAND)
    e0, e1, e2 = _sc_gather(idx_all, t0, t1, t2)
    return _tc_mlp(sb, e0, e1, e2, W1, b1, W2, b2)


# speedup vs baseline: 1.0204x; 1.0204x over previous
import functools

import jax
import jax.numpy as jnp
from jax import lax
from jax.experimental import pallas as pl
from jax.experimental.pallas import tpu as pltpu
from jax.experimental.pallas import tpu_sc as plsc

B = 16384
D = 64
DP = 2 * D

_info = plsc.get_sparse_core_info()
_NC, _NS = _info.num_cores, _info.num_subcores
_NW = _NC * _NS
_BPW = B // _NW
_CHUNK = 128
_NCHUNK = _BPW // _CHUNK
_HALFC = _NCHUNK // 2
_IROWS = 16

_mesh = plsc.VectorSubcoreMesh(core_axis_name="c", subcore_axis_name="s")


def _pair_transpose(tT, C):
    cols = tT.shape[1]
    nblk = (cols + C - 1) // C

    S = 4096 if (C // 2) % 4096 == 0 else C // 2

    def body(t_ref, out_ref):
        for b in range(C // 2 // S):
            lo, hi = b * S, (b + 1) * S
            trl = jnp.transpose(t_ref[:, lo:hi])
            trr = jnp.transpose(t_ref[:, C // 2 + lo:C // 2 + hi])
            out_ref[lo:hi, :] = jnp.concatenate([trl, trr], axis=1)

    return pl.pallas_call(
        body,
        grid=(nblk,),
        in_specs=[pl.BlockSpec((D, C), lambda i: (0, i))],
        out_specs=pl.BlockSpec((C // 2, 128), lambda i: (i, 0)),
        out_shape=jax.ShapeDtypeStruct((nblk * C // 2, 128), jnp.float32),
    )(tT)


def _pair_index(r, C):
    half = C // 2
    p = (r // C) * half + (r % half)
    side = (r // half) % 2
    return p, side


@functools.partial(
    pl.kernel,
    mesh=_mesh,
    compiler_params=pltpu.CompilerParams(use_tc_tiling_on_sc=True),
    out_type=[jax.ShapeDtypeStruct((B, DP), jnp.float32)] * 3,
    scratch_types=[
        pltpu.VMEM((_IROWS, _CHUNK), jnp.int32),
        pltpu.VMEM((_HALFC * _CHUNK, DP), jnp.float32),
        pltpu.VMEM((_HALFC * _CHUNK, DP), jnp.float32),
        pltpu.VMEM((_HALFC * _CHUNK, DP), jnp.float32),
        pltpu.SemaphoreType.DMA,
        pltpu.SemaphoreType.DMA,
        pltpu.SemaphoreType.DMA,
    ],
)
def _sc_gather(idx_all, t0, t1, t2, o0, o1, o2,
               iv, r0, r1, r2, s0, s1, s2):
    wid = lax.axis_index("s") * _NC + lax.axis_index("c")
    base = wid * _BPW
    pltpu.sync_copy(idx_all.at[wid], iv)
    for h in range(2):
        copies = []
        for j in range(_HALFC):
            c = h * _HALFC + j
            dst = pl.ds(j * _CHUNK, _CHUNK)
            copies.append(pltpu.async_copy(t0.at[iv.at[c]], r0.at[dst], s0))
            copies.append(
                pltpu.async_copy(t1.at[iv.at[_NCHUNK + c]], r1.at[dst], s1))
            copies.append(
                pltpu.async_copy(t2.at[iv.at[2 * _NCHUNK + c]], r2.at[dst], s2))
        for c in copies:
            c.wait()
        span = pl.ds(base + h * _HALFC * _CHUNK, _HALFC * _CHUNK)
        pltpu.sync_copy(r0, o0.at[span])
        pltpu.sync_copy(r1, o1.at[span])
        pltpu.sync_copy(r2, o2.at[span])


_BM = 2048


def _mlp_body(sb, e0, e1, e2, w1, b1, w2, b2, out):
    side = sb[...]
    sel = []
    for k, e in enumerate((e0, e1, e2)):
        p = (side[:, k] == 1).reshape(-1, 1)
        sel.append(jnp.where(p, e[:, D:], e[:, :D]))
    emb = jnp.concatenate(sel, axis=1)
    h = jnp.dot(emb, w1[...], preferred_element_type=jnp.float32) + b1[...]
    h = jnp.maximum(h, 0.0)
    out[...] = jnp.dot(h, w2[...], preferred_element_type=jnp.float32) + b2[...]


def _tc_mlp(sb, e0, e1, e2, W1, b1, W2, b2):
    n1, n2 = W1.shape[1], W2.shape[1]
    grid = (B // _BM,)
    return pl.pallas_call(
        _mlp_body,
        grid=grid,
        in_specs=[
            pl.BlockSpec((_BM, 3), lambda i: (i, 0)),
            pl.BlockSpec((_BM, DP), lambda i: (i, 0)),
            pl.BlockSpec((_BM, DP), lambda i: (i, 0)),
            pl.BlockSpec((_BM, DP), lambda i: (i, 0)),
            pl.BlockSpec((3 * D, n1), lambda i: (0, 0)),
            pl.BlockSpec((1, n1), lambda i: (0, 0)),
            pl.BlockSpec((n1, n2), lambda i: (0, 0)),
            pl.BlockSpec((1, n2), lambda i: (0, 0)),
        ],
        out_specs=pl.BlockSpec((_BM, n2), lambda i: (i, 0)),
        out_shape=jax.ShapeDtypeStruct((B, n2), jnp.float32),
    )(sb, e0, e1, e2, W1, b1.reshape(1, n1), W2, b2.reshape(1, n2))


_C_ITEM = 32768
_C_CAT = 1000
_C_BRAND = 10000


def kernel(x, table_item_id, table_category, table_brand, W1, b1, W2, b2):
    xi = x.astype(jnp.int32)
    p0, s0 = _pair_index(xi[:, 0], _C_ITEM)
    p1, s1 = _pair_index(xi[:, 1], _C_CAT)
    p2, s2 = _pair_index(xi[:, 2], _C_BRAND)
    idx0 = p0.reshape(_NW, _NCHUNK, _CHUNK)
    idx1 = p1.reshape(_NW, _NCHUNK, _CHUNK)
    idx2 = p2.reshape(_NW, _NCHUNK, _CHUNK)
    pad = jnp.zeros((_NW, _IROWS - 3 * _NCHUNK, _CHUNK), jnp.int32)
    idx_all = jnp.concatenate([idx0, idx1, idx2, pad], axis=1)
    sb = jnp.stack([s0, s1, s2], axis=1)
    t0 = _pair_transpose(table_item_id.T, _C_ITEM)
    t1 = _pair_transpose(table_category.T, _C_CAT)
    t2 = _pair_transpose(table_brand.T, _C_BRAND)
    e0, e1, e2 = _sc_gather(idx_all, t0, t1, t2)
    return _tc_mlp(sb, e0, e1, e2, W1, b1, W2, b2)


# speedup vs baseline: 1.0312x; 1.0106x over previous
import functools

import jax
import jax.numpy as jnp
from jax import lax
from jax.experimental import pallas as pl
from jax.experimental.pallas import tpu as pltpu
from jax.experimental.pallas import tpu_sc as plsc

B = 16384
D = 64
DP = 2 * D

_info = plsc.get_sparse_core_info()
_NC, _NS = _info.num_cores, _info.num_subcores
_NW = _NC * _NS
_BPW = B // _NW
_CHUNK = 128
_NCHUNK = _BPW // _CHUNK
_HALFC = _NCHUNK // 2
_IROWS = 16

_mesh = plsc.VectorSubcoreMesh(core_axis_name="c", subcore_axis_name="s")


def _pair_transpose(tT, C):
    cols = tT.shape[1]
    nblk = (cols + C - 1) // C

    S = 4096 if (C // 2) % 4096 == 0 else C // 2

    def body(t_ref, out_ref):
        for b in range(C // 2 // S):
            lo, hi = b * S, (b + 1) * S
            trl = jnp.transpose(t_ref[:, lo:hi])
            trr = jnp.transpose(t_ref[:, C // 2 + lo:C // 2 + hi])
            out_ref[lo:hi, :] = jnp.concatenate([trl, trr], axis=1)

    return pl.pallas_call(
        body,
        grid=(nblk,),
        in_specs=[pl.BlockSpec((D, C), lambda i: (0, i))],
        out_specs=pl.BlockSpec((C // 2, 128), lambda i: (i, 0)),
        out_shape=jax.ShapeDtypeStruct((nblk * C // 2, 128), jnp.float32),
    )(tT)


def _pair_index(r, C):
    half = C // 2
    p = (r // C) * half + (r % half)
    side = (r // half) % 2
    return p, side


@functools.partial(
    pl.kernel,
    mesh=_mesh,
    compiler_params=pltpu.CompilerParams(use_tc_tiling_on_sc=True),
    out_type=[jax.ShapeDtypeStruct((B, DP), jnp.float32)] * 3,
    scratch_types=[
        pltpu.VMEM((_IROWS, _CHUNK), jnp.int32),
        pltpu.VMEM((_HALFC * _CHUNK, DP), jnp.float32),
        pltpu.VMEM((_HALFC * _CHUNK, DP), jnp.float32),
        pltpu.VMEM((_HALFC * _CHUNK, DP), jnp.float32),
        pltpu.SemaphoreType.DMA,
        pltpu.SemaphoreType.DMA,
        pltpu.SemaphoreType.DMA,
    ],
)
def _sc_gather(idx_all, t0, t1, t2, o0, o1, o2,
               iv, r0, r1, r2, s0, s1, s2):
    wid = lax.axis_index("s") * _NC + lax.axis_index("c")
    base = wid * _BPW
    pltpu.sync_copy(idx_all.at[wid], iv)
    for h in range(2):
        copies = []
        for j in range(_HALFC):
            c = h * _HALFC + j
            dst = pl.ds(j * _CHUNK, _CHUNK)
            copies.append(pltpu.async_copy(t0.at[iv.at[c]], r0.at[dst], s0))
            copies.append(
                pltpu.async_copy(t1.at[iv.at[_NCHUNK + c]], r1.at[dst], s1))
            copies.append(
                pltpu.async_copy(t2.at[iv.at[2 * _NCHUNK + c]], r2.at[dst], s2))
        for c in copies:
            c.wait()
        span = pl.ds(base + h * _HALFC * _CHUNK, _HALFC * _CHUNK)
        pltpu.sync_copy(r0, o0.at[span])
        pltpu.sync_copy(r1, o1.at[span])
        pltpu.sync_copy(r2, o2.at[span])


_BM = 2048


def _mlp_body(sb, e0, e1, e2, w1, b1, w2, b2, out):
    side = sb[...].astype(jnp.int32)
    sel = []
    for k, e in enumerate((e0, e1, e2)):
        p = (side[:, k] == 1).reshape(-1, 1)
        sel.append(jnp.where(p, e[:, D:], e[:, :D]))
    emb = jnp.concatenate(sel, axis=1)
    h = jnp.dot(emb, w1[...], preferred_element_type=jnp.float32) + b1[...]
    h = jnp.maximum(h, 0.0)
    out[...] = jnp.dot(h, w2[...], preferred_element_type=jnp.float32) + b2[...]


def _tc_mlp(sb, e0, e1, e2, W1, b1, W2, b2):
    n1, n2 = W1.shape[1], W2.shape[1]
    grid = (B // _BM,)
    return pl.pallas_call(
        _mlp_body,
        grid=grid,
        in_specs=[
            pl.BlockSpec((_BM, 3), lambda i: (i, 0)),
            pl.BlockSpec((_BM, DP), lambda i: (i, 0)),
            pl.BlockSpec((_BM, DP), lambda i: (i, 0)),
            pl.BlockSpec((_BM, DP), lambda i: (i, 0)),
            pl.BlockSpec((3 * D, n1), lambda i: (0, 0)),
            pl.BlockSpec((1, n1), lambda i: (0, 0)),
            pl.BlockSpec((n1, n2), lambda i: (0, 0)),
            pl.BlockSpec((1, n2), lambda i: (0, 0)),
        ],
        out_specs=pl.BlockSpec((_BM, n2), lambda i: (i, 0)),
        out_shape=jax.ShapeDtypeStruct((B, n2), jnp.float32),
    )(sb, e0, e1, e2, W1, b1.reshape(1, n1), W2, b2.reshape(1, n2))


_C_ITEM = 32768
_C_CAT = 1000
_C_BRAND = 10000


def kernel(x, table_item_id, table_category, table_brand, W1, b1, W2, b2):
    xi = x.astype(jnp.int32)
    p0, s0 = _pair_index(xi[:, 0], _C_ITEM)
    p1, s1 = _pair_index(xi[:, 1], _C_CAT)
    p2, s2 = _pair_index(xi[:, 2], _C_BRAND)
    idx0 = p0.reshape(_NW, _NCHUNK, _CHUNK)
    idx1 = p1.reshape(_NW, _NCHUNK, _CHUNK)
    idx2 = p2.reshape(_NW, _NCHUNK, _CHUNK)
    pad = jnp.zeros((_NW, _IROWS - 3 * _NCHUNK, _CHUNK), jnp.int32)
    idx_all = jnp.concatenate([idx0, idx1, idx2, pad], axis=1)
    sb = jnp.stack([s0, s1, s2], axis=1).astype(jnp.int8)
    t0 = _pair_transpose(table_item_id.T, _C_ITEM)
    t1 = _pair_transpose(table_category.T, _C_CAT)
    t2 = _pair_transpose(table_brand.T, _C_BRAND)
    e0, e1, e2 = _sc_gather(idx_all, t0, t1, t2)
    return _tc_mlp(sb, e0, e1, e2, W1, b1, W2, b2)
